# Initial kernel scaffold; baseline (speedup 1.0000x reference)
import jax
import jax.numpy as jnp
from jax.experimental import pallas as pl


def kernel(x, rd, node_in_W1, node_in_b1, node_in_W2, node_in_b2, msg_W1, msg_b1, msg_W2, msg_b2, upd_W1, upd_b1, upd_W2, upd_b2, rd_W1, rd_b1, rd_W2, rd_b2, gs_W1, gs_b1, gs_W2, gs_b2, head_W1, head_b1, head_W2, head_b2, edge_index, batch_ids, n_nodes_g, n_edges_g):
    raise NotImplementedError("write your pallas kernel here")


# speedup vs baseline: 2.9700x; 2.9700x over previous
import jax
import jax.numpy as jnp
from jax import lax
from jax.experimental import pallas as pl
from jax.experimental.pallas import tpu as pltpu
from jax.experimental.pallas import tpu_sc as plsc

N = 10000
E = 320000
G = 64
DH = 128
NRBF = 16
L = 4
YD = 4

NC = 2
NS = 16
NW = NC * NS
EPW = E // NW
EPWP = 10240
EPAD = NW * EPWP
NPAD = 10240
RPT = NPAD // NS

BE = 128
NBLK = EPWP // BE
RCH = 128
NCHK = RPT // RCH

BE_E = 64
NBLK_E = EPWP // BE_E
RCH_E = 64
NCHK_E = RPT // RCH_E

_MESH = plsc.VectorSubcoreMesh(
    core_axis_name="c", subcore_axis_name="s", num_cores=NC, num_subcores=NS)
_SC_PARAMS = pltpu.CompilerParams(needs_layout_passes=False)


def _sc_prep_body(x0_h, x1_h, srcf_h, dstf_h, d2_h,
                  x0_v, x1_v, srcf_v, dstf_v, d2_v):
    cid = lax.axis_index("c")
    sid = lax.axis_index("s")
    wid = cid * NS + sid
    pltpu.sync_copy(x0_h, x0_v)
    pltpu.sync_copy(x1_h, x1_v)
    pltpu.sync_copy(srcf_h.at[pl.ds(wid * EPWP, EPWP)], srcf_v)
    pltpu.sync_copy(dstf_h.at[pl.ds(wid * EPWP, EPWP)], dstf_v)

    def d2i(i, _):
        sv = srcf_v[pl.ds(i * 16, 16)]
        dv = dstf_v[pl.ds(i * 16, 16)]
        dx = plsc.load_gather(x0_v, [sv]) - plsc.load_gather(x0_v, [dv])
        dy = plsc.load_gather(x1_v, [sv]) - plsc.load_gather(x1_v, [dv])
        d2_v[pl.ds(i * 16, 16)] = dx * dx + dy * dy + 1e-8
        return 0

    lax.fori_loop(0, EPWP // 16, d2i, 0)
    pltpu.sync_copy(d2_v, d2_h.at[pl.ds(wid * EPWP, EPWP)])


_sc_prep = pl.kernel(
    _sc_prep_body,
    out_type=jax.ShapeDtypeStruct((EPAD,), jnp.float32),
    mesh=_MESH,
    scratch_types=[
        pltpu.VMEM((NPAD,), jnp.float32),
        pltpu.VMEM((NPAD,), jnp.float32),
        pltpu.VMEM((EPWP,), jnp.int32),
        pltpu.VMEM((EPWP,), jnp.int32),
        pltpu.VMEM((EPWP,), jnp.float32),
    ],
    compiler_params=_SC_PARAMS,
)


def _sc_deg_body(dstm_h, deg_h, dstm_v, ones_v, deg_sh):
    cid = lax.axis_index("c")
    sid = lax.axis_index("s")
    wid = cid * NS + sid
    pltpu.sync_copy(dstm_h.at[wid], dstm_v)

    zeros16 = jnp.zeros((16,), jnp.float32)
    ones16 = jnp.full((16,), 1.0, jnp.float32)

    def fill0(r, _):
        for k in range(8):
            ones_v[r, pl.ds(k * 16, 16)] = zeros16
        return 0

    lax.fori_loop(0, RCH, fill0, 0)
    for t in range(NCHK):
        pltpu.sync_copy(ones_v, deg_sh.at[pl.ds(sid * RPT + t * RCH, RCH)])

    def fill1(r, _):
        for k in range(8):
            ones_v[r, pl.ds(k * 16, 16)] = ones16
        return 0

    lax.fori_loop(0, RCH, fill1, 0)
    plsc.subcore_barrier()

    def degblk(j, _):
        pltpu.sync_copy(ones_v, deg_sh.at[dstm_v.at[j]], add=True)
        return 0

    lax.fori_loop(0, NBLK, degblk, 0)

    plsc.subcore_barrier()
    for t in range(NCHK):
        pltpu.sync_copy(deg_sh.at[pl.ds(sid * RPT + t * RCH, RCH)], ones_v)
        pltpu.sync_copy(ones_v, deg_h.at[cid, pl.ds(sid * RPT + t * RCH, RCH)])


_sc_deg = pl.kernel(
    _sc_deg_body,
    out_type=jax.ShapeDtypeStruct((NC, NPAD, DH), jnp.float32),
    mesh=_MESH,
    scratch_types=[
        pltpu.VMEM((NBLK, BE), jnp.int32),
        pltpu.VMEM((RCH, DH), jnp.float32),
        pltpu.VMEM_SHARED((NPAD, DH), jnp.float32),
    ],
    compiler_params=_SC_PARAMS,
)


def _sc_edge_body(p_h, q_h, r_h, srcf_h, dstf_h, out_h,
                  srcf_v, dstf_v, pbuf, qbuf, rbuf, a_sh, sem):
    cid = lax.axis_index("c")
    sid = lax.axis_index("s")
    wid = cid * NS + sid
    pltpu.sync_copy(srcf_h.at[pl.ds(wid * EPWP, EPWP)], srcf_v)
    pltpu.sync_copy(dstf_h.at[pl.ds(wid * EPWP, EPWP)], dstf_v)

    zeros16 = jnp.zeros((16,), jnp.float32)

    def zrow(r, _):
        for k in range(8):
            pbuf[r, pl.ds(k * 16, 16)] = zeros16
        return 0

    lax.fori_loop(0, RCH_E, zrow, 0)
    for t in range(NCHK_E):
        pltpu.sync_copy(pbuf, a_sh.at[pl.ds(sid * RPT + t * RCH_E, RCH_E)])
    plsc.subcore_barrier()

    def blk(j, _):
        copies = []
        for k in range(BE_E // 16):
            sv = srcf_v[pl.ds(j * BE_E + k * 16, 16)]
            dv = dstf_v[pl.ds(j * BE_E + k * 16, 16)]
            rows = pl.ds(k * 16, 16)
            copies.append(pltpu.async_copy(p_h.at[sv], pbuf.at[rows], sem))
            copies.append(pltpu.async_copy(q_h.at[dv], qbuf.at[rows], sem))
        copies.append(pltpu.async_copy(r_h.at[wid * NBLK_E + j], rbuf, sem))
        for c in copies:
            c.wait()

        def crow(r, _):
            for k in range(8):
                s = pl.ds(k * 16, 16)
                v = pbuf[r, s] + qbuf[r, s] + rbuf[r, s]
                pbuf[r, s] = jnp.maximum(v, 0.0)
            return 0

        lax.fori_loop(0, BE_E, crow, 0)

        for k in range(BE_E // 16):
            dv = dstf_v[pl.ds(j * BE_E + k * 16, 16)]
            pltpu.sync_copy(pbuf.at[pl.ds(k * 16, 16)], a_sh.at[dv],
                            add=True)
        return 0

    lax.fori_loop(0, NBLK_E, blk, 0)

    plsc.subcore_barrier()
    for t in range(NCHK_E):
        pltpu.sync_copy(a_sh.at[pl.ds(sid * RPT + t * RCH_E, RCH_E)], pbuf)
        pltpu.sync_copy(pbuf, out_h.at[cid, pl.ds(sid * RPT + t * RCH_E,
                                                  RCH_E)])


_sc_edge = pl.kernel(
    _sc_edge_body,
    out_type=jax.ShapeDtypeStruct((NC, NPAD, DH), jnp.float32),
    mesh=_MESH,
    scratch_types=[
        pltpu.VMEM((EPWP,), jnp.int32),
        pltpu.VMEM((EPWP,), jnp.int32),
        pltpu.VMEM((BE_E, DH), jnp.float32),
        pltpu.VMEM((BE_E, DH), jnp.float32),
        pltpu.VMEM((BE_E, DH), jnp.float32),
        pltpu.VMEM_SHARED((NPAD, DH), jnp.float32),
        pltpu.SemaphoreType.DMA,
    ],
    compiler_params=_SC_PARAMS,
)


_NBN = 10
_BN = N // _NBN
_RBLK = 2048
_RGRID = EPAD // _RBLK


def _tc_node_body(x_ref, w1, b1, w2, b2, o_ref):
    h = jnp.maximum(
        jnp.dot(x_ref[...], w1[...], preferred_element_type=jnp.float32)
        + b1[...], 0.0)
    o_ref[...] = (jnp.dot(h, w2[...], preferred_element_type=jnp.float32)
                  + b2[...])


_tc_node = pl.pallas_call(
    _tc_node_body,
    grid=(_NBN,),
    in_specs=[
        pl.BlockSpec((_BN, 2), lambda i: (i, 0)),
        pl.BlockSpec((2, DH), lambda i: (0, 0)),
        pl.BlockSpec((1, DH), lambda i: (0, 0)),
        pl.BlockSpec((DH, DH), lambda i: (0, 0)),
        pl.BlockSpec((1, DH), lambda i: (0, 0)),
    ],
    out_specs=pl.BlockSpec((_BN, DH), lambda i: (i, 0)),
    out_shape=jax.ShapeDtypeStruct((N, DH), jnp.float32),
)


def _tc_rbf_body(d2_ref, c_ref, w_ref, o0, o1, o2, o3):
    dist = jnp.sqrt(d2_ref[...])
    diff = dist - c_ref[...]
    rbf = jnp.exp(-10.0 * diff * diff)
    for l, o in enumerate((o0, o1, o2, o3)):
        o[...] = (jnp.dot(rbf, w_ref[l, :NRBF, :],
                          preferred_element_type=jnp.float32)
                  + w_ref[l, NRBF:NRBF + 1, :])


_tc_rbf = pl.pallas_call(
    _tc_rbf_body,
    grid=(_RGRID,),
    in_specs=[
        pl.BlockSpec((_RBLK, 1), lambda i: (i, 0)),
        pl.BlockSpec((1, NRBF), lambda i: (0, 0)),
        pl.BlockSpec((L, NRBF + 1, DH), lambda i: (0, 0, 0)),
    ],
    out_specs=[pl.BlockSpec((_RBLK, DH), lambda i: (i, 0))] * L,
    out_shape=[jax.ShapeDtypeStruct((EPAD, DH), jnp.float32)] * L,
)


def _tc_pq_body(h_ref, wa, wb, wu, p_ref, q_ref, hu_ref):
    h = h_ref[...]
    p_ref[...] = jnp.dot(h, wa[...], preferred_element_type=jnp.float32)
    q_ref[...] = jnp.dot(h, wb[...], preferred_element_type=jnp.float32)
    hu_ref[...] = jnp.dot(h, wu[...], preferred_element_type=jnp.float32)


_tc_pq = pl.pallas_call(
    _tc_pq_body,
    grid=(_NBN,),
    in_specs=[
        pl.BlockSpec((_BN, DH), lambda i: (i, 0)),
        pl.BlockSpec((DH, DH), lambda i: (0, 0)),
        pl.BlockSpec((DH, DH), lambda i: (0, 0)),
        pl.BlockSpec((DH, DH), lambda i: (0, 0)),
    ],
    out_specs=[pl.BlockSpec((_BN, DH), lambda i: (i, 0))] * 3,
    out_shape=[jax.ShapeDtypeStruct((NPAD, DH), jnp.float32)] * 3,
)


def _tc_upd_body(a_ref, hu_ref, h_ref, dg_ref, w2, b2, u1b, c1, u2, c2,
                 o_ref):
    asum = a_ref[0] + a_ref[1]
    deg = dg_ref[0, :, 0:1] + dg_ref[1, :, 0:1]
    w2r = w2[...].astype(jnp.bfloat16).astype(jnp.float32)
    agg = (jnp.dot(asum, w2r, preferred_element_type=jnp.float32,
                   precision=lax.Precision.HIGHEST)
           + deg * b2[...])
    pre = (hu_ref[...]
           + jnp.dot(agg, u1b[...], preferred_element_type=jnp.float32)
           + c1[...])
    u = (jnp.dot(jnp.maximum(pre, 0.0), u2[...],
                 preferred_element_type=jnp.float32) + c2[...])
    o_ref[...] = h_ref[...] + u


_tc_upd = pl.pallas_call(
    _tc_upd_body,
    grid=(_NBN,),
    in_specs=[
        pl.BlockSpec((NC, _BN, DH), lambda i: (0, i, 0)),
        pl.BlockSpec((_BN, DH), lambda i: (i, 0)),
        pl.BlockSpec((_BN, DH), lambda i: (i, 0)),
        pl.BlockSpec((NC, _BN, DH), lambda i: (0, i, 0)),
        pl.BlockSpec((DH, DH), lambda i: (0, 0)),
        pl.BlockSpec((1, DH), lambda i: (0, 0)),
        pl.BlockSpec((DH, DH), lambda i: (0, 0)),
        pl.BlockSpec((1, DH), lambda i: (0, 0)),
        pl.BlockSpec((DH, DH), lambda i: (0, 0)),
        pl.BlockSpec((1, DH), lambda i: (0, 0)),
    ],
    out_specs=pl.BlockSpec((_BN, DH), lambda i: (i, 0)),
    out_shape=jax.ShapeDtypeStruct((N, DH), jnp.float32),
)


def _tc_pool_body(h_ref, bid_ref, rd_ref, nn_ref, ne_ref,
                  rw1, rb1, rw2, rb2, gw1, gb1, gw2, gb2,
                  hw, hb1, hw2, hb2, o_ref, mx_ref):
    h = h_ref[...]
    bid = bid_ref[...]
    gid = lax.broadcasted_iota(jnp.int32, (N, G), 1)
    onehot = (bid == gid).astype(jnp.float32)
    dn = (((0,), (0,)), ((), ()))
    msum = lax.dot_general(onehot, h, dn, preferred_element_type=jnp.float32,
                           precision=lax.Precision.HIGHEST)
    cnt = lax.dot_general(onehot, jnp.ones((N, 1), jnp.float32), dn,
                          preferred_element_type=jnp.float32,
                          precision=lax.Precision.HIGHEST)
    mean = msum / jnp.maximum(cnt, 1.0)

    neg = jnp.float32(-jnp.inf)

    def mx_g(g, c):
        m = jnp.where(bid == g, h, neg)
        mg = jnp.max(m, axis=0, keepdims=True)
        mx_ref[pl.ds(g, 1), :] = mg
        return c

    lax.fori_loop(0, G, mx_g, 0)
    mx = mx_ref[...]
    mx = jnp.where(jnp.isfinite(mx), mx, 0.0)

    rdh = jnp.maximum(rd_ref[...] * rw1[...] + rb1[...], 0.0)
    rdh = jnp.dot(rdh, rw2[...], preferred_element_type=jnp.float32) + rb2[...]

    sh = jnp.maximum(jnp.log(1.0 + nn_ref[...]) * gw1[0:1, :]
                     + jnp.log(1.0 + ne_ref[...]) * gw1[1:2, :]
                     + gb1[...], 0.0)
    sh = jnp.dot(sh, gw2[...], preferred_element_type=jnp.float32) + gb2[...]

    z = (jnp.dot(mean, hw[0], preferred_element_type=jnp.float32)
         + jnp.dot(mx, hw[1], preferred_element_type=jnp.float32)
         + jnp.dot(rdh, hw[2], preferred_element_type=jnp.float32)
         + jnp.dot(sh, hw[3], preferred_element_type=jnp.float32)
         + hb1[...])
    o_ref[...] = (jnp.dot(jnp.maximum(z, 0.0), hw2[...],
                          preferred_element_type=jnp.float32) + hb2[...])


_tc_pool = pl.pallas_call(
    _tc_pool_body,
    in_specs=[
        pl.BlockSpec((N, DH), lambda: (0, 0)),
        pl.BlockSpec((N, 1), lambda: (0, 0)),
        pl.BlockSpec((G, 1), lambda: (0, 0)),
        pl.BlockSpec((G, 1), lambda: (0, 0)),
        pl.BlockSpec((G, 1), lambda: (0, 0)),
        pl.BlockSpec((1, DH), lambda: (0, 0)),
        pl.BlockSpec((1, DH), lambda: (0, 0)),
        pl.BlockSpec((DH, DH), lambda: (0, 0)),
        pl.BlockSpec((1, DH), lambda: (0, 0)),
        pl.BlockSpec((2, DH), lambda: (0, 0)),
        pl.BlockSpec((1, DH), lambda: (0, 0)),
        pl.BlockSpec((DH, DH), lambda: (0, 0)),
        pl.BlockSpec((1, DH), lambda: (0, 0)),
        pl.BlockSpec((4, DH, DH), lambda: (0, 0, 0)),
        pl.BlockSpec((1, DH), lambda: (0, 0)),
        pl.BlockSpec((DH, YD), lambda: (0, 0)),
        pl.BlockSpec((1, YD), lambda: (0, 0)),
    ],
    out_specs=pl.BlockSpec((G, YD), lambda: (0, 0)),
    out_shape=jax.ShapeDtypeStruct((G, YD), jnp.float32),
    scratch_shapes=[pltpu.VMEM((G, DH), jnp.float32)],
)


def kernel(x, rd, node_in_W1, node_in_b1, node_in_W2, node_in_b2,
           msg_W1, msg_b1, msg_W2, msg_b2, upd_W1, upd_b1, upd_W2, upd_b2,
           rd_W1, rd_b1, rd_W2, rd_b2, gs_W1, gs_b1, gs_W2, gs_b2,
           head_W1, head_b1, head_W2, head_b2,
           edge_index, batch_ids, n_nodes_g, n_edges_g):
    src = edge_index[0].astype(jnp.int32).reshape(NW, EPW)
    dst = edge_index[1].astype(jnp.int32).reshape(NW, EPW)
    src = jnp.pad(src, ((0, 0), (0, EPWP - EPW)), constant_values=N)
    dst = jnp.pad(dst, ((0, 0), (0, EPWP - EPW)), constant_values=N)
    dstm = dst.reshape(NW, NBLK, BE)
    srcf = src.reshape(NW * EPWP)
    dstf = dst.reshape(NW * EPWP)
    x0 = jnp.pad(x[:, 0], (0, NPAD - N))
    x1 = jnp.pad(x[:, 1], (0, NPAD - N))

    d2 = _sc_prep(x0, x1, srcf, dstf)
    deg2 = _sc_deg(dstm)

    centers = jnp.linspace(0.0, 4.0, NRBF).astype(jnp.float32)
    w1c_aug = jnp.concatenate(
        [msg_W1[:, 2 * DH:, :], msg_b1[:, None, :]], axis=1)
    rs = _tc_rbf(d2.reshape(EPAD, 1), centers.reshape(1, NRBF), w1c_aug)

    h = _tc_node(x, node_in_W1, node_in_b1.reshape(1, DH),
                 node_in_W2, node_in_b2.reshape(1, DH))

    for l in range(L):
        p, q, hu = _tc_pq(h, msg_W1[l, :DH, :], msg_W1[l, DH:2 * DH, :],
                          upd_W1[l, :DH, :])
        a2 = _sc_edge(p, q, rs[l].reshape(EPAD // BE_E, BE_E, DH),
                      srcf, dstf)
        h = _tc_upd(a2, hu, h, deg2, msg_W2[l], msg_b2[l].reshape(1, DH),
                    upd_W1[l, DH:, :], upd_b1[l].reshape(1, DH),
                    upd_W2[l], upd_b2[l].reshape(1, DH))

    out = _tc_pool(
        h, batch_ids.astype(jnp.int32).reshape(N, 1), rd,
        n_nodes_g.astype(jnp.float32).reshape(G, 1),
        n_edges_g.astype(jnp.float32).reshape(G, 1),
        rd_W1.reshape(1, DH), rd_b1.reshape(1, DH), rd_W2,
        rd_b2.reshape(1, DH),
        gs_W1, gs_b1.reshape(1, DH), gs_W2, gs_b2.reshape(1, DH),
        head_W1.reshape(4, DH, DH), head_b1.reshape(1, DH),
        head_W2, head_b2.reshape(1, YD))
    return out
